# Initial kernel scaffold; baseline (speedup 1.0000x reference)
import functools

import jax
import jax.numpy as jnp
from jax import lax
from jax.experimental import pallas as pl
from jax.experimental.pallas import tpu as pltpu
from jax.experimental.pallas import tpu_sc as plsc

B = 4096
C = 256
T = 512
DEPTH = 8
N_INT = 2**DEPTH - 1
N_LEAF = 2**DEPTH
NC = 2
NS = 16
NW = NC * NS
BPW = B // NW
LANES = 16
TCHUNK = 16
NCHUNK = T // TCHUNK


def _forest_body(x_hbm, th_hbm, or_hbm, w_hbm, out_hbm,
                 x_v, th_v, or_v, w_v, out_v):
    wid = lax.axis_index("s") * NC + lax.axis_index("c")
    b0 = wid * BPW
    pltpu.sync_copy(x_hbm.at[pl.ds(b0, BPW)], x_v)

    lane_iota = lax.iota(jnp.int32, LANES)
    ones = jnp.ones((LANES,), jnp.int32)
    zeros = jnp.zeros((LANES,), jnp.int32)

    def chunk_body(c, carry):
        t0 = c * TCHUNK
        pltpu.sync_copy(th_hbm.at[pl.ds(t0, TCHUNK)], th_v)
        pltpu.sync_copy(or_hbm.at[pl.ds(t0, TCHUNK)], or_v)
        pltpu.sync_copy(w_hbm.at[pl.ds(t0, TCHUNK)], w_v)

        def tree_body(tc, carry2):
            tcv = jnp.full((LANES,), tc, jnp.int32)
            tfull = jnp.full((LANES,), t0 + tc, jnp.int32)
            for i in range(BPW // LANES):
                lanes = lane_iota + (i * LANES)
                node = zeros
                mab = jnp.full((LANES,), jnp.inf, jnp.float32)
                for _ in range(DEPTH):
                    th = plsc.load_gather(th_v, [tcv, node])
                    od = plsc.load_gather(or_v, [tcv, node])
                    ft = plsc.load_gather(x_v, [lanes, od])
                    m = ft - th
                    mab = jnp.minimum(mab, jnp.abs(m))
                    node = 2 * node + 1 + jnp.where(m > 0, ones, zeros)
                leaf = node - N_INT
                w = plsc.load_gather(w_v, [tcv, leaf])
                plsc.store_scatter(out_v, [lanes, tfull], w * mab)
            return carry2

        lax.fori_loop(0, TCHUNK, tree_body, 0)
        return carry

    lax.fori_loop(0, NCHUNK, chunk_body, 0)
    pltpu.sync_copy(out_v, out_hbm.at[pl.ds(b0, BPW)])


@jax.jit
def _forest(x, th_pad, or_pad, weights):
    mesh = plsc.VectorSubcoreMesh(core_axis_name="c", subcore_axis_name="s")
    fwd = functools.partial(
        pl.kernel,
        mesh=mesh,
        out_type=jax.ShapeDtypeStruct((B, T), jnp.float32),
        scratch_types=[
            pltpu.VMEM((BPW, C), jnp.float32),
            pltpu.VMEM((TCHUNK, N_LEAF), jnp.float32),
            pltpu.VMEM((TCHUNK, N_LEAF), jnp.int32),
            pltpu.VMEM((TCHUNK, N_LEAF), jnp.float32),
            pltpu.VMEM((BPW, T), jnp.float32),
        ],
    )(_forest_body)
    return fwd(x, th_pad, or_pad, weights)


def kernel(x, thresholds, weights, ordinals):
    th_pad = jnp.pad(thresholds, ((0, 0), (0, 1)))
    or_pad = jnp.pad(ordinals, ((0, 0), (0, 1)))
    return _forest(x, th_pad, or_pad, weights)


# speedup vs baseline: 752.4999x; 752.4999x over previous
import functools

import jax
import jax.numpy as jnp
from jax import lax
from jax.experimental import pallas as pl
from jax.experimental.pallas import tpu as pltpu
from jax.experimental.pallas import tpu_sc as plsc

B = 4096
C = 256
T = 512
DEPTH = 8
N_INT = 2**DEPTH - 1
N_LEAF = 2**DEPTH
NC = 2
NS = 16
NW = NC * NS
BPW = B // NW
LANES = 16
TCHUNK = 16
NCHUNK = T // TCHUNK


def _forest_body(x_hbm, th_hbm, or_hbm, w_hbm, out_hbm,
                 x_v, th_v, or_v, w_v, out_v):
    wid = lax.axis_index("s") * NC + lax.axis_index("c")
    b0 = wid * BPW
    pltpu.sync_copy(x_hbm.at[pl.ds(b0 * C, BPW * C)], x_v)

    lane_iota = lax.iota(jnp.int32, LANES)
    ones = jnp.ones((LANES,), jnp.int32)
    zeros = jnp.zeros((LANES,), jnp.int32)

    def chunk_body(c, carry):
        t0 = c * TCHUNK
        pltpu.sync_copy(th_hbm.at[pl.ds(t0 * N_LEAF, TCHUNK * N_LEAF)], th_v)
        pltpu.sync_copy(or_hbm.at[pl.ds(t0 * N_LEAF, TCHUNK * N_LEAF)], or_v)
        pltpu.sync_copy(w_hbm.at[pl.ds(t0 * N_LEAF, TCHUNK * N_LEAF)], w_v)

        def tree_body(tc, carry2):
            tbase = jnp.full((LANES,), tc * N_LEAF, jnp.int32)
            obase = lane_iota * T + (t0 + tc)
            for i in range(BPW // LANES):
                xbase = (lane_iota + (i * LANES)) * C
                node = zeros
                mab = jnp.full((LANES,), jnp.inf, jnp.float32)
                for _ in range(DEPTH):
                    tnode = tbase + node
                    th = plsc.load_gather(th_v, [tnode])
                    od = plsc.load_gather(or_v, [tnode])
                    ft = plsc.load_gather(x_v, [xbase + od])
                    m = ft - th
                    mab = jnp.minimum(mab, jnp.abs(m))
                    node = 2 * node + 1 + jnp.where(m > 0, ones, zeros)
                leaf = node - N_INT
                w = plsc.load_gather(w_v, [tbase + leaf])
                plsc.store_scatter(out_v, [obase + (i * LANES * T)], w * mab)
            return carry2

        lax.fori_loop(0, TCHUNK, tree_body, 0)
        return carry

    lax.fori_loop(0, NCHUNK, chunk_body, 0)
    pltpu.sync_copy(out_v, out_hbm.at[pl.ds(b0 * T, BPW * T)])


@jax.jit
def _forest(x, th_pad, or_pad, weights):
    mesh = plsc.VectorSubcoreMesh(core_axis_name="c", subcore_axis_name="s")
    fwd = functools.partial(
        pl.kernel,
        mesh=mesh,
        compiler_params=pltpu.CompilerParams(
            use_tc_tiling_on_sc=False, needs_layout_passes=False),
        out_type=jax.ShapeDtypeStruct((B * T,), jnp.float32),
        scratch_types=[
            pltpu.VMEM((BPW * C,), jnp.float32),
            pltpu.VMEM((TCHUNK * N_LEAF,), jnp.float32),
            pltpu.VMEM((TCHUNK * N_LEAF,), jnp.int32),
            pltpu.VMEM((TCHUNK * N_LEAF,), jnp.float32),
            pltpu.VMEM((BPW * T,), jnp.float32),
        ],
    )(_forest_body)
    out = fwd(x.reshape(-1), th_pad.reshape(-1), or_pad.reshape(-1),
              weights.reshape(-1))
    return out.reshape(B, T)


def kernel(x, thresholds, weights, ordinals):
    th_pad = jnp.pad(thresholds, ((0, 0), (0, 1)))
    or_pad = jnp.pad(ordinals, ((0, 0), (0, 1)))
    return _forest(x, th_pad, or_pad, weights)


# speedup vs baseline: 1600.0974x; 2.1264x over previous
import functools

import jax
import jax.numpy as jnp
from jax import lax
from jax.experimental import pallas as pl
from jax.experimental.pallas import tpu as pltpu
from jax.experimental.pallas import tpu_sc as plsc

B = 4096
C = 256
T = 512
DEPTH = 8
N_INT = 2**DEPTH - 1
N_LEAF = 2**DEPTH
NC = 2
NS = 16
NW = NC * NS
BPW = B // NW
LANES = 16
TCHUNK = 16
NCHUNK = T // TCHUNK


def _forest_body(x_hbm, th_hbm, or_hbm, w_hbm, out_hbm,
                 x_v, th_v, or_v, w_v, out_v):
    wid = lax.axis_index("s") * NC + lax.axis_index("c")
    b0 = wid * BPW
    pltpu.sync_copy(x_hbm.at[pl.ds(b0 * C, BPW * C)], x_v)

    lane_iota = lax.iota(jnp.int32, LANES)
    ones = jnp.ones((LANES,), jnp.int32)
    zeros = jnp.zeros((LANES,), jnp.int32)

    def chunk_body(c, carry):
        t0 = c * TCHUNK
        pltpu.sync_copy(th_hbm.at[pl.ds(t0 * N_LEAF, TCHUNK * N_LEAF)], th_v)
        pltpu.sync_copy(or_hbm.at[pl.ds(t0 * N_LEAF, TCHUNK * N_LEAF)], or_v)
        pltpu.sync_copy(w_hbm.at[pl.ds(t0 * N_LEAF, TCHUNK * N_LEAF)], w_v)

        def tree_body(tc, carry2):
            NV = BPW // LANES
            tbase = jnp.full((LANES,), tc * N_LEAF, jnp.int32)
            obase = lane_iota * T + (t0 + tc)
            xbases = [(lane_iota + (i * LANES)) * C for i in range(NV)]
            nodes = [zeros] * NV
            mabs = [jnp.full((LANES,), jnp.inf, jnp.float32)] * NV
            for _ in range(DEPTH):
                tns = [tbase + nodes[i] for i in range(NV)]
                ths = [plsc.load_gather(th_v, [tns[i]]) for i in range(NV)]
                ods = [plsc.load_gather(or_v, [tns[i]]) for i in range(NV)]
                fts = [plsc.load_gather(x_v, [xbases[i] + ods[i]])
                       for i in range(NV)]
                for i in range(NV):
                    m = fts[i] - ths[i]
                    mabs[i] = jnp.minimum(mabs[i], jnp.abs(m))
                    nodes[i] = 2 * nodes[i] + 1 + jnp.where(m > 0, ones, zeros)
            ws = [plsc.load_gather(w_v, [tbase + (nodes[i] - N_INT)])
                  for i in range(NV)]
            for i in range(NV):
                plsc.store_scatter(out_v, [obase + (i * LANES * T)],
                                   ws[i] * mabs[i])
            return carry2

        lax.fori_loop(0, TCHUNK, tree_body, 0)
        return carry

    lax.fori_loop(0, NCHUNK, chunk_body, 0)
    pltpu.sync_copy(out_v, out_hbm.at[pl.ds(b0 * T, BPW * T)])


@jax.jit
def _forest(x, th_pad, or_pad, weights):
    mesh = plsc.VectorSubcoreMesh(core_axis_name="c", subcore_axis_name="s")
    fwd = functools.partial(
        pl.kernel,
        mesh=mesh,
        compiler_params=pltpu.CompilerParams(
            use_tc_tiling_on_sc=False, needs_layout_passes=False),
        out_type=jax.ShapeDtypeStruct((B * T,), jnp.float32),
        scratch_types=[
            pltpu.VMEM((BPW * C,), jnp.float32),
            pltpu.VMEM((TCHUNK * N_LEAF,), jnp.float32),
            pltpu.VMEM((TCHUNK * N_LEAF,), jnp.int32),
            pltpu.VMEM((TCHUNK * N_LEAF,), jnp.float32),
            pltpu.VMEM((BPW * T,), jnp.float32),
        ],
    )(_forest_body)
    out = fwd(x.reshape(-1), th_pad.reshape(-1), or_pad.reshape(-1),
              weights.reshape(-1))
    return out.reshape(B, T)


def kernel(x, thresholds, weights, ordinals):
    th_pad = jnp.pad(thresholds, ((0, 0), (0, 1)))
    or_pad = jnp.pad(ordinals, ((0, 0), (0, 1)))
    return _forest(x, th_pad, or_pad, weights)


# speedup vs baseline: 1921.3332x; 1.2008x over previous
import functools

import jax
import jax.numpy as jnp
from jax import lax
from jax.experimental import pallas as pl
from jax.experimental.pallas import tpu as pltpu
from jax.experimental.pallas import tpu_sc as plsc

B = 4096
C = 256
T = 512
DEPTH = 8
N_INT = 2**DEPTH - 1
N_LEAF = 2**DEPTH
NC = 2
NS = 16
NW = NC * NS
BPW = B // NW
LANES = 16
TCHUNK = 16
NCHUNK = T // TCHUNK


def _forest_body(x_hbm, th_hbm, or_hbm, w_hbm, out_hbm,
                 x_v, th_a, or_a, w_a, th_b, or_b, w_b, out_v,
                 sem_a, sem_b):
    wid = lax.axis_index("s") * NC + lax.axis_index("c")
    b0 = wid * BPW
    pltpu.sync_copy(x_hbm.at[pl.ds(b0 * C, BPW * C)], x_v)

    lane_iota = lax.iota(jnp.int32, LANES)
    ones = jnp.ones((LANES,), jnp.int32)
    zeros = jnp.zeros((LANES,), jnp.int32)

    def fetch(c, th_v, or_v, w_v, sem):
        off = c * (TCHUNK * N_LEAF)
        pltpu.async_copy(th_hbm.at[pl.ds(off, TCHUNK * N_LEAF)], th_v, sem)
        pltpu.async_copy(or_hbm.at[pl.ds(off, TCHUNK * N_LEAF)], or_v, sem)
        pltpu.async_copy(w_hbm.at[pl.ds(off, TCHUNK * N_LEAF)], w_v, sem)

    def drain(th_v, or_v, w_v, sem):
        pltpu.make_async_copy(
            th_hbm.at[pl.ds(0, TCHUNK * N_LEAF)], th_v, sem).wait()
        pltpu.make_async_copy(
            or_hbm.at[pl.ds(0, TCHUNK * N_LEAF)], or_v, sem).wait()
        pltpu.make_async_copy(
            w_hbm.at[pl.ds(0, TCHUNK * N_LEAF)], w_v, sem).wait()

    def compute_chunk(c, th_v, or_v, w_v):
        t0 = c * TCHUNK

        def tree_body(tc, carry2):
            NV = BPW // LANES
            tbase = jnp.full((LANES,), tc * N_LEAF, jnp.int32)
            obase = lane_iota * T + (t0 + tc)
            xbases = [(lane_iota + (i * LANES)) * C for i in range(NV)]
            nodes = [zeros] * NV
            mabs = [jnp.full((LANES,), jnp.inf, jnp.float32)] * NV
            for _ in range(DEPTH):
                tns = [tbase + nodes[i] for i in range(NV)]
                ths = [plsc.load_gather(th_v, [tns[i]]) for i in range(NV)]
                ods = [plsc.load_gather(or_v, [tns[i]]) for i in range(NV)]
                fts = [plsc.load_gather(x_v, [xbases[i] + ods[i]])
                       for i in range(NV)]
                for i in range(NV):
                    m = fts[i] - ths[i]
                    mabs[i] = jnp.minimum(mabs[i], jnp.abs(m))
                    nodes[i] = 2 * nodes[i] + 1 + jnp.where(m > 0, ones, zeros)
            ws = [plsc.load_gather(w_v, [tbase + (nodes[i] - N_INT)])
                  for i in range(NV)]
            for i in range(NV):
                plsc.store_scatter(out_v, [obase + (i * LANES * T)],
                                   ws[i] * mabs[i])
            return carry2

        lax.fori_loop(0, TCHUNK, tree_body, 0)

    fetch(0, th_a, or_a, w_a, sem_a)
    fetch(1, th_b, or_b, w_b, sem_b)

    def pair_body(i, carry):
        c = 2 * i
        drain(th_a, or_a, w_a, sem_a)
        compute_chunk(c, th_a, or_a, w_a)
        fetch(jnp.minimum(c + 2, NCHUNK - 1), th_a, or_a, w_a, sem_a)
        drain(th_b, or_b, w_b, sem_b)
        compute_chunk(c + 1, th_b, or_b, w_b)
        fetch(jnp.minimum(c + 3, NCHUNK - 1), th_b, or_b, w_b, sem_b)
        return carry

    lax.fori_loop(0, NCHUNK // 2, pair_body, 0)
    drain(th_a, or_a, w_a, sem_a)
    drain(th_b, or_b, w_b, sem_b)
    pltpu.sync_copy(out_v, out_hbm.at[pl.ds(b0 * T, BPW * T)])


@jax.jit
def _forest(x, th_pad, or_pad, weights):
    mesh = plsc.VectorSubcoreMesh(core_axis_name="c", subcore_axis_name="s")
    fwd = functools.partial(
        pl.kernel,
        mesh=mesh,
        compiler_params=pltpu.CompilerParams(
            use_tc_tiling_on_sc=False, needs_layout_passes=False),
        out_type=jax.ShapeDtypeStruct((B * T,), jnp.float32),
        scratch_types=[
            pltpu.VMEM((BPW * C,), jnp.float32),
            pltpu.VMEM((TCHUNK * N_LEAF,), jnp.float32),
            pltpu.VMEM((TCHUNK * N_LEAF,), jnp.int32),
            pltpu.VMEM((TCHUNK * N_LEAF,), jnp.float32),
            pltpu.VMEM((TCHUNK * N_LEAF,), jnp.float32),
            pltpu.VMEM((TCHUNK * N_LEAF,), jnp.int32),
            pltpu.VMEM((TCHUNK * N_LEAF,), jnp.float32),
            pltpu.VMEM((BPW * T,), jnp.float32),
            pltpu.SemaphoreType.DMA,
            pltpu.SemaphoreType.DMA,
        ],
    )(_forest_body)
    out = fwd(x.reshape(-1), th_pad.reshape(-1), or_pad.reshape(-1),
              weights.reshape(-1))
    return out.reshape(B, T)


def kernel(x, thresholds, weights, ordinals):
    th_pad = jnp.pad(thresholds, ((0, 0), (0, 1)))
    or_pad = jnp.pad(ordinals, ((0, 0), (0, 1)))
    return _forest(x, th_pad, or_pad, weights)


# speedup vs baseline: 3704.0096x; 1.9278x over previous
import functools

import jax
import jax.numpy as jnp
from jax import lax
from jax.experimental import pallas as pl
from jax.experimental.pallas import tpu as pltpu
from jax.experimental.pallas import tpu_sc as plsc

B = 4096
C = 256
CP = 257
T = 512
DEPTH = 8
N_INT = 2**DEPTH - 1
N_LEAF = 2**DEPTH
NC = 2
NS = 16
NW = NC * NS
BPW = B // NW
LANES = 16
NV = BPW // LANES
TCHUNK = 16
NCHUNK = T // TCHUNK


def _forest_body(x_hbm, th_hbm, or_hbm, w_hbm, out_hbm,
                 x_v, th_a, or_a, w_a, th_b, or_b, w_b, out_v,
                 sem_a, sem_b):
    wid = lax.axis_index("s") * NC + lax.axis_index("c")
    b0 = wid * BPW
    pltpu.sync_copy(x_hbm.at[pl.ds(b0 * CP, BPW * CP)], x_v)

    lane_iota = lax.iota(jnp.int32, LANES)
    zeros = jnp.zeros((LANES,), jnp.int32)
    xbases = [(lane_iota + (i * LANES)) * CP for i in range(NV)]

    def fetch(c, th_v, or_v, w_v, sem):
        off = c * (TCHUNK * N_LEAF)
        pltpu.async_copy(th_hbm.at[pl.ds(off, TCHUNK * N_LEAF)], th_v, sem)
        pltpu.async_copy(or_hbm.at[pl.ds(off, TCHUNK * N_LEAF)], or_v, sem)
        pltpu.async_copy(w_hbm.at[pl.ds(off, TCHUNK * N_LEAF)], w_v, sem)

    def drain(th_v, or_v, w_v, sem):
        pltpu.make_async_copy(
            th_hbm.at[pl.ds(0, TCHUNK * N_LEAF)], th_v, sem).wait()
        pltpu.make_async_copy(
            or_hbm.at[pl.ds(0, TCHUNK * N_LEAF)], or_v, sem).wait()
        pltpu.make_async_copy(
            w_hbm.at[pl.ds(0, TCHUNK * N_LEAF)], w_v, sem).wait()

    def compute_chunk(c, th_v, or_v, w_v):
        t0 = c * TCHUNK

        def tree_body(tc, carry2):
            tb = tc * N_LEAF
            obase = lane_iota * T + (t0 + tc)
            cv0 = zeros + (1 - tb)
            cv1 = cv0 + 1
            nas = [zeros + tb] * NV
            mabs = [jnp.full((LANES,), jnp.inf, jnp.float32)] * NV
            for _ in range(DEPTH):
                ths = [plsc.load_gather(th_v, [nas[i]]) for i in range(NV)]
                ods = [plsc.load_gather(or_v, [nas[i]]) for i in range(NV)]
                fts = [plsc.load_gather(x_v, [xbases[i] + ods[i]])
                       for i in range(NV)]
                for i in range(NV):
                    m = fts[i] - ths[i]
                    mabs[i] = jnp.minimum(mabs[i], jnp.abs(m))
                    nas[i] = (nas[i] + nas[i]) + jnp.where(m > 0, cv1, cv0)
            ws = [plsc.load_gather(w_v, [nas[i] - N_INT]) for i in range(NV)]
            for i in range(NV):
                plsc.store_scatter(out_v, [obase + (i * LANES * T)],
                                   ws[i] * mabs[i])
            return carry2

        lax.fori_loop(0, TCHUNK, tree_body, 0)

    fetch(0, th_a, or_a, w_a, sem_a)
    fetch(1, th_b, or_b, w_b, sem_b)

    def pair_body(i, carry):
        c = 2 * i
        drain(th_a, or_a, w_a, sem_a)
        compute_chunk(c, th_a, or_a, w_a)
        fetch(jnp.minimum(c + 2, NCHUNK - 1), th_a, or_a, w_a, sem_a)
        drain(th_b, or_b, w_b, sem_b)
        compute_chunk(c + 1, th_b, or_b, w_b)
        fetch(jnp.minimum(c + 3, NCHUNK - 1), th_b, or_b, w_b, sem_b)
        return carry

    lax.fori_loop(0, NCHUNK // 2, pair_body, 0)
    drain(th_a, or_a, w_a, sem_a)
    drain(th_b, or_b, w_b, sem_b)
    pltpu.sync_copy(out_v, out_hbm.at[pl.ds(b0 * T, BPW * T)])


@jax.jit
def _forest(x, th_pad, or_pad, weights):
    mesh = plsc.VectorSubcoreMesh(core_axis_name="c", subcore_axis_name="s")
    fwd = functools.partial(
        pl.kernel,
        mesh=mesh,
        compiler_params=pltpu.CompilerParams(
            use_tc_tiling_on_sc=False, needs_layout_passes=False),
        out_type=jax.ShapeDtypeStruct((B * T,), jnp.float32),
        scratch_types=[
            pltpu.VMEM((BPW * CP,), jnp.float32),
            pltpu.VMEM((TCHUNK * N_LEAF,), jnp.float32),
            pltpu.VMEM((TCHUNK * N_LEAF,), jnp.int32),
            pltpu.VMEM((TCHUNK * N_LEAF,), jnp.float32),
            pltpu.VMEM((TCHUNK * N_LEAF,), jnp.float32),
            pltpu.VMEM((TCHUNK * N_LEAF,), jnp.int32),
            pltpu.VMEM((TCHUNK * N_LEAF,), jnp.float32),
            pltpu.VMEM((BPW * T,), jnp.float32),
            pltpu.SemaphoreType.DMA,
            pltpu.SemaphoreType.DMA,
        ],
    )(_forest_body)
    out = fwd(x.reshape(-1), th_pad.reshape(-1), or_pad.reshape(-1),
              weights.reshape(-1))
    return out.reshape(B, T)


def kernel(x, thresholds, weights, ordinals):
    th_pad = jnp.pad(thresholds, ((0, 0), (0, 1)))
    or_pad = jnp.pad(ordinals, ((0, 0), (0, 1)))
    x_pad = jnp.pad(x, ((0, 0), (0, CP - C)))
    return _forest(x_pad, th_pad, or_pad, weights)


# speedup vs baseline: 3762.1751x; 1.0157x over previous
import functools

import jax
import jax.numpy as jnp
from jax import lax
from jax.experimental import pallas as pl
from jax.experimental.pallas import tpu as pltpu
from jax.experimental.pallas import tpu_sc as plsc

B = 4096
C = 256
CP = 257
T = 512
DEPTH = 8
N_INT = 2**DEPTH - 1
N_LEAF = 2**DEPTH
NC = 2
NS = 16
NW = NC * NS
BPW = B // NW
LANES = 16
NV = BPW // LANES
TCHUNK = 16
NCHUNK = T // TCHUNK


def _forest_body(x_hbm, th_hbm, or_hbm, w_hbm, out_hbm,
                 x_v, th_a, or_a, w_a, th_b, or_b, w_b, out_v,
                 sem_a, sem_b):
    wid = lax.axis_index("s") * NC + lax.axis_index("c")
    b0 = wid * BPW
    pltpu.sync_copy(x_hbm.at[pl.ds(b0 * CP, BPW * CP)], x_v)

    lane_iota = lax.iota(jnp.int32, LANES)
    zeros = jnp.zeros((LANES,), jnp.int32)
    xbases = [(lane_iota + (i * LANES)) * CP for i in range(NV)]

    def fetch(c, th_v, or_v, w_v, sem):
        off = c * (TCHUNK * N_LEAF)
        pltpu.async_copy(th_hbm.at[pl.ds(off, TCHUNK * N_LEAF)], th_v, sem)
        pltpu.async_copy(or_hbm.at[pl.ds(off, TCHUNK * N_LEAF)], or_v, sem)
        pltpu.async_copy(w_hbm.at[pl.ds(off, TCHUNK * N_LEAF)], w_v, sem)

    def drain(th_v, or_v, w_v, sem):
        pltpu.make_async_copy(
            th_hbm.at[pl.ds(0, TCHUNK * N_LEAF)], th_v, sem).wait()
        pltpu.make_async_copy(
            or_hbm.at[pl.ds(0, TCHUNK * N_LEAF)], or_v, sem).wait()
        pltpu.make_async_copy(
            w_hbm.at[pl.ds(0, TCHUNK * N_LEAF)], w_v, sem).wait()

    def compute_chunk(c, th_v, or_v, w_v):
        t0 = c * TCHUNK

        def tree_body(tc, carry2):
            tb = tc * N_LEAF
            obase = lane_iota * T + (t0 + tc)
            cv0 = zeros + (1 - tb)
            cv1 = cv0 + 1
            cw0 = cv0 - N_INT
            cw1 = cw0 + 1
            k1 = zeros + (tb + 1)
            k2 = k1 + 1
            th0 = plsc.load_gather(th_v, [zeros + tb])
            od0 = plsc.load_gather(or_v, [zeros + tb])
            fts = [plsc.load_gather(x_v, [xbases[i] + od0])
                   for i in range(NV)]
            ms = [fts[i] - th0 for i in range(NV)]
            mabs = [jnp.abs(ms[i]) for i in range(NV)]
            nas = [jnp.where(ms[i] > 0, k2, k1) for i in range(NV)]
            for _ in range(DEPTH - 2):
                ths = [plsc.load_gather(th_v, [nas[i]]) for i in range(NV)]
                ods = [plsc.load_gather(or_v, [nas[i]]) for i in range(NV)]
                fts = [plsc.load_gather(x_v, [xbases[i] + ods[i]])
                       for i in range(NV)]
                for i in range(NV):
                    m = fts[i] - ths[i]
                    mabs[i] = jnp.minimum(mabs[i], jnp.abs(m))
                    nas[i] = (nas[i] + nas[i]) + jnp.where(m > 0, cv1, cv0)
            ths = [plsc.load_gather(th_v, [nas[i]]) for i in range(NV)]
            ods = [plsc.load_gather(or_v, [nas[i]]) for i in range(NV)]
            fts = [plsc.load_gather(x_v, [xbases[i] + ods[i]])
                   for i in range(NV)]
            for i in range(NV):
                m = fts[i] - ths[i]
                mabs[i] = jnp.minimum(mabs[i], jnp.abs(m))
                nas[i] = (nas[i] + nas[i]) + jnp.where(m > 0, cw1, cw0)
            ws = [plsc.load_gather(w_v, [nas[i]]) for i in range(NV)]
            for i in range(NV):
                plsc.store_scatter(out_v, [obase + (i * LANES * T)],
                                   ws[i] * mabs[i])
            return carry2

        lax.fori_loop(0, TCHUNK, tree_body, 0)

    fetch(0, th_a, or_a, w_a, sem_a)
    fetch(1, th_b, or_b, w_b, sem_b)

    def pair_body(i, carry):
        c = 2 * i
        drain(th_a, or_a, w_a, sem_a)
        compute_chunk(c, th_a, or_a, w_a)
        fetch(jnp.minimum(c + 2, NCHUNK - 1), th_a, or_a, w_a, sem_a)
        drain(th_b, or_b, w_b, sem_b)
        compute_chunk(c + 1, th_b, or_b, w_b)
        fetch(jnp.minimum(c + 3, NCHUNK - 1), th_b, or_b, w_b, sem_b)
        return carry

    lax.fori_loop(0, NCHUNK // 2, pair_body, 0)
    drain(th_a, or_a, w_a, sem_a)
    drain(th_b, or_b, w_b, sem_b)
    pltpu.sync_copy(out_v, out_hbm.at[pl.ds(b0 * T, BPW * T)])


@jax.jit
def _forest(x, th_pad, or_pad, weights):
    mesh = plsc.VectorSubcoreMesh(core_axis_name="c", subcore_axis_name="s")
    fwd = functools.partial(
        pl.kernel,
        mesh=mesh,
        compiler_params=pltpu.CompilerParams(
            use_tc_tiling_on_sc=False, needs_layout_passes=False),
        out_type=jax.ShapeDtypeStruct((B * T,), jnp.float32),
        scratch_types=[
            pltpu.VMEM((BPW * CP,), jnp.float32),
            pltpu.VMEM((TCHUNK * N_LEAF,), jnp.float32),
            pltpu.VMEM((TCHUNK * N_LEAF,), jnp.int32),
            pltpu.VMEM((TCHUNK * N_LEAF,), jnp.float32),
            pltpu.VMEM((TCHUNK * N_LEAF,), jnp.float32),
            pltpu.VMEM((TCHUNK * N_LEAF,), jnp.int32),
            pltpu.VMEM((TCHUNK * N_LEAF,), jnp.float32),
            pltpu.VMEM((BPW * T,), jnp.float32),
            pltpu.SemaphoreType.DMA,
            pltpu.SemaphoreType.DMA,
        ],
    )(_forest_body)
    out = fwd(x.reshape(-1), th_pad.reshape(-1), or_pad.reshape(-1),
              weights.reshape(-1))
    return out.reshape(B, T)


def kernel(x, thresholds, weights, ordinals):
    th_pad = jnp.pad(thresholds, ((0, 0), (0, 1)))
    or_pad = jnp.pad(ordinals, ((0, 0), (0, 1)))
    x_pad = jnp.pad(x, ((0, 0), (0, CP - C)))
    return _forest(x_pad, th_pad, or_pad, weights)


# speedup vs baseline: 4643.0500x; 1.2341x over previous
import functools

import jax
import jax.numpy as jnp
from jax import lax
from jax.experimental import pallas as pl
from jax.experimental.pallas import tpu as pltpu
from jax.experimental.pallas import tpu_sc as plsc

B = 4096
C = 256
CP = 257
T = 512
DEPTH = 8
N_INT = 2**DEPTH - 1
N_LEAF = 2**DEPTH
NC = 2
NS = 16
NW = NC * NS
BPW = B // NW
LANES = 16
NV = BPW // LANES
TCHUNK = 16
NCHUNK = T // TCHUNK


def _forest_body(x_hbm, th_hbm, or_hbm, w_hbm, out_hbm,
                 x_v, th_a, or_a, w_a, th_b, or_b, w_b, out_v,
                 sem_a, sem_b):
    wid = lax.axis_index("s") * NC + lax.axis_index("c")
    b0 = wid * BPW
    pltpu.sync_copy(x_hbm.at[pl.ds(b0 * CP, BPW * CP)], x_v)

    lane_iota = lax.iota(jnp.int32, LANES)
    zeros = jnp.zeros((LANES,), jnp.int32)
    xbases = [(lane_iota + (i * LANES)) * CP for i in range(NV)]

    def fetch(c, th_v, or_v, w_v, sem):
        off = c * (TCHUNK * N_LEAF)
        pltpu.async_copy(th_hbm.at[pl.ds(off, TCHUNK * N_LEAF)], th_v, sem)
        pltpu.async_copy(or_hbm.at[pl.ds(off, TCHUNK * N_LEAF)], or_v, sem)
        pltpu.async_copy(w_hbm.at[pl.ds(off, TCHUNK * N_LEAF)], w_v, sem)

    def drain(th_v, or_v, w_v, sem):
        pltpu.make_async_copy(
            th_hbm.at[pl.ds(0, TCHUNK * N_LEAF)], th_v, sem).wait()
        pltpu.make_async_copy(
            or_hbm.at[pl.ds(0, TCHUNK * N_LEAF)], or_v, sem).wait()
        pltpu.make_async_copy(
            w_hbm.at[pl.ds(0, TCHUNK * N_LEAF)], w_v, sem).wait()

    def compute_chunk(c, th_v, or_v, w_v):
        t0 = c * TCHUNK

        def tree_body(tc, carry2):
            tb = tc * N_LEAF
            obase = lane_iota + (t0 + tc) * BPW
            cv0 = zeros + (1 - tb)
            cv1 = cv0 + 1
            cw0 = cv0 - N_INT
            cw1 = cw0 + 1
            k1 = zeros + (tb + 1)
            k2 = k1 + 1
            th0 = plsc.load_gather(th_v, [zeros + tb])
            od0 = plsc.load_gather(or_v, [zeros + tb])
            fts = [plsc.load_gather(x_v, [xbases[i] + od0])
                   for i in range(NV)]
            ms = [fts[i] - th0 for i in range(NV)]
            mabs = [jnp.abs(ms[i]) for i in range(NV)]
            nas = [jnp.where(ms[i] > 0, k2, k1) for i in range(NV)]
            for _ in range(DEPTH - 2):
                ths = [plsc.load_gather(th_v, [nas[i]]) for i in range(NV)]
                ods = [plsc.load_gather(or_v, [nas[i]]) for i in range(NV)]
                fts = [plsc.load_gather(x_v, [xbases[i] + ods[i]])
                       for i in range(NV)]
                for i in range(NV):
                    m = fts[i] - ths[i]
                    mabs[i] = jnp.minimum(mabs[i], jnp.abs(m))
                    nas[i] = (nas[i] + nas[i]) + jnp.where(m > 0, cv1, cv0)
            ths = [plsc.load_gather(th_v, [nas[i]]) for i in range(NV)]
            ods = [plsc.load_gather(or_v, [nas[i]]) for i in range(NV)]
            fts = [plsc.load_gather(x_v, [xbases[i] + ods[i]])
                   for i in range(NV)]
            for i in range(NV):
                m = fts[i] - ths[i]
                mabs[i] = jnp.minimum(mabs[i], jnp.abs(m))
                nas[i] = (nas[i] + nas[i]) + jnp.where(m > 0, cw1, cw0)
            ws = [plsc.load_gather(w_v, [nas[i]]) for i in range(NV)]
            for i in range(NV):
                plsc.store_scatter(out_v, [obase + (i * LANES)],
                                   ws[i] * mabs[i])
            return carry2

        lax.fori_loop(0, TCHUNK, tree_body, 0)

    fetch(0, th_a, or_a, w_a, sem_a)
    fetch(1, th_b, or_b, w_b, sem_b)

    def pair_body(i, carry):
        c = 2 * i
        drain(th_a, or_a, w_a, sem_a)
        compute_chunk(c, th_a, or_a, w_a)
        fetch(jnp.minimum(c + 2, NCHUNK - 1), th_a, or_a, w_a, sem_a)
        drain(th_b, or_b, w_b, sem_b)
        compute_chunk(c + 1, th_b, or_b, w_b)
        fetch(jnp.minimum(c + 3, NCHUNK - 1), th_b, or_b, w_b, sem_b)
        return carry

    lax.fori_loop(0, NCHUNK // 2, pair_body, 0)
    drain(th_a, or_a, w_a, sem_a)
    drain(th_b, or_b, w_b, sem_b)
    pltpu.sync_copy(out_v, out_hbm.at[pl.ds(b0 * T, BPW * T)])


@jax.jit
def _forest(x, th_pad, or_pad, weights):
    mesh = plsc.VectorSubcoreMesh(core_axis_name="c", subcore_axis_name="s")
    fwd = functools.partial(
        pl.kernel,
        mesh=mesh,
        compiler_params=pltpu.CompilerParams(
            use_tc_tiling_on_sc=False, needs_layout_passes=False),
        out_type=jax.ShapeDtypeStruct((B * T,), jnp.float32),
        scratch_types=[
            pltpu.VMEM((BPW * CP,), jnp.float32),
            pltpu.VMEM((TCHUNK * N_LEAF,), jnp.float32),
            pltpu.VMEM((TCHUNK * N_LEAF,), jnp.int32),
            pltpu.VMEM((TCHUNK * N_LEAF,), jnp.float32),
            pltpu.VMEM((TCHUNK * N_LEAF,), jnp.float32),
            pltpu.VMEM((TCHUNK * N_LEAF,), jnp.int32),
            pltpu.VMEM((TCHUNK * N_LEAF,), jnp.float32),
            pltpu.VMEM((BPW * T,), jnp.float32),
            pltpu.SemaphoreType.DMA,
            pltpu.SemaphoreType.DMA,
        ],
    )(_forest_body)
    out = fwd(x.reshape(-1), th_pad.reshape(-1), or_pad.reshape(-1),
              weights.reshape(-1))
    return out.reshape(NW, T, BPW).transpose(0, 2, 1).reshape(B, T)


def kernel(x, thresholds, weights, ordinals):
    th_pad = jnp.pad(thresholds, ((0, 0), (0, 1)))
    or_pad = jnp.pad(ordinals, ((0, 0), (0, 1)))
    x_pad = jnp.pad(x, ((0, 0), (0, CP - C)))
    return _forest(x_pad, th_pad, or_pad, weights)
